# baseline (speedup 1.0000x reference)
import jax
import jax.numpy as jnp
from jax.experimental import pallas as pl
from jax.experimental.pallas import tpu as pltpu

E = 8
D = 2048
DFF = 4096
T = 256
NF = 8
BF = DFF // NF
NS = E * NF


def _moe_kernel(xT_ref, wr_ref, br_ref, w1_ref, w2_ref, w3_ref,
                b1_ref, b2_ref, b3_ref, out_ref,
                wrow_ref, xb_ref, h_ref):
    s = pl.program_id(0)

    @pl.when(s == 0)
    def _init():
        logits = jnp.dot(wr_ref[...], xT_ref[...],
                         preferred_element_type=jnp.float32) + br_ref[...]
        idx = jax.lax.broadcasted_iota(jnp.int32, (E, T), 0)
        m1 = jnp.max(logits, axis=0, keepdims=True)
        i1 = jnp.min(jnp.where(logits == m1, idx, E), axis=0, keepdims=True)
        sel1 = idx == i1
        masked = jnp.where(sel1, -jnp.inf, logits)
        m2 = jnp.max(masked, axis=0, keepdims=True)
        i2 = jnp.min(jnp.where(masked == m2, idx, E), axis=0, keepdims=True)
        sel2 = idx == i2
        p1 = 1.0 / (1.0 + jnp.exp(m2 - m1))
        wrow_ref[...] = jnp.where(sel1, p1, 0.0) + jnp.where(sel2, 1.0 - p1, 0.0)
        xb_ref[...] = xT_ref[...].astype(jnp.bfloat16)

    @pl.when(s < NS)
    def _h_phase():
        f = s % NF
        xb = xb_ref[...]
        b1f = b1_ref[0, pl.ds(f * BF, BF), :]
        b2f = b2_ref[0, pl.ds(f * BF, BF), :]
        h1 = jnp.dot(w1_ref[0], xb, preferred_element_type=jnp.float32) + b1f
        h2 = jnp.dot(w2_ref[0], xb, preferred_element_type=jnp.float32) + b2f
        h = h2 * (h1 * jax.nn.sigmoid(h1))
        h_ref[s % 2] = h.astype(jnp.bfloat16)

    @pl.when(s > 0)
    def _y_phase():
        sp = s - 1
        ep = sp // NF
        fp = sp % NF
        hp = h_ref[(s - 1) % 2]
        yblk = jnp.dot(w3_ref[0], hp, preferred_element_type=jnp.float32)
        wrow = wrow_ref[pl.ds(ep, 1), :]
        contrib = yblk * wrow

        @pl.when(fp == 0)
        def _b3():
            contrib2 = contrib + b3_ref[0] * wrow

            @pl.when(ep == 0)
            def _set():
                out_ref[...] = contrib2

            @pl.when(ep > 0)
            def _acc():
                out_ref[...] += contrib2

        @pl.when(fp > 0)
        def _nob3():
            out_ref[...] += contrib


def kernel(x, Wr, br, W1, b1, W2, b2, W3, b3):
    b, s_, d = x.shape
    xT = x.reshape(b * s_, d).T
    last = NS - 1

    def w12_map(s):
        sc = jnp.minimum(s, last)
        return (sc // NF, sc % NF, 0)

    def w3_map(s):
        sp = jnp.maximum(s - 1, 0)
        return (sp // NF, 0, sp % NF)

    outT = pl.pallas_call(
        _moe_kernel,
        grid=(NS + 1,),
        in_specs=[
            pl.BlockSpec((D, T), lambda s: (0, 0)),
            pl.BlockSpec((E, D), lambda s: (0, 0)),
            pl.BlockSpec((E, 1), lambda s: (0, 0)),
            pl.BlockSpec((1, BF, D), w12_map),
            pl.BlockSpec((1, BF, D), w12_map),
            pl.BlockSpec((1, D, BF), w3_map),
            pl.BlockSpec((1, DFF, 1),
                         lambda s: (jnp.minimum(s, last) // NF, 0, 0)),
            pl.BlockSpec((1, DFF, 1),
                         lambda s: (jnp.minimum(s, last) // NF, 0, 0)),
            pl.BlockSpec((1, D, 1),
                         lambda s: (jnp.maximum(s - 1, 0) // NF, 0, 0)),
        ],
        out_specs=pl.BlockSpec((D, T), lambda s: (0, 0)),
        out_shape=jax.ShapeDtypeStruct((D, T), jnp.float32),
        scratch_shapes=[
            pltpu.VMEM((E, T), jnp.float32),
            pltpu.VMEM((D, T), jnp.bfloat16),
            pltpu.VMEM((2, BF, T), jnp.bfloat16),
        ],
    )(xT, Wr, br.reshape(E, 1), W1, W2, W3,
      b1.reshape(E, DFF, 1), b2.reshape(E, DFF, 1), b3.reshape(E, D, 1))
    return outT.T.reshape(b, s_, d)
